# baseline (speedup 1.0000x reference)
import functools

import jax
import jax.numpy as jnp
from jax import lax
from jax.experimental import pallas as pl
from jax.experimental.pallas import tpu as pltpu
from jax.experimental.pallas import tpu_sc as plsc

_N = 8192
_NT = 16
_CHUNK = _N // _NT
_L = 16
_NV = _CHUNK // _L

_CLASS = 42
_MIN = 0.1
_MAX = 1.0
_MATCHED_POINT = 1.5


@functools.partial(
    pl.kernel,
    out_type=jax.ShapeDtypeStruct((_NT * _L,), jnp.float32),
    mesh=plsc.VectorSubcoreMesh(
        core_axis_name="c", subcore_axis_name="s", num_cores=1
    ),
    scratch_types=[
        pltpu.VMEM((_CHUNK + _L,), jnp.float32),
        pltpu.SemaphoreType.DMA,
    ],
)
def _judge_sc(score_hbm, out_hbm, buf_v, sem):
    sid = lax.axis_index("s")
    base = sid * _CHUNK

    cp = pltpu.async_copy(
        score_hbm.at[0, pl.ds(base, _CHUNK)], buf_v.at[pl.ds(0, _CHUNK)], sem)
    pltpu.sync_copy(score_hbm.at[0, pl.ds(40, _L)],
                    buf_v.at[pl.ds(_CHUNK, _L)])

    lane = lax.iota(jnp.int32, _L)
    cv = buf_v[pl.ds(_CHUNK, _L)]
    cb = cv.at[jnp.full((_L,), _CLASS - 40, jnp.int32)].get(
        mode="promise_in_bounds")

    cp.wait()

    one = jnp.full((_L,), 1.0, jnp.float32)
    zero = jnp.zeros((_L,), jnp.float32)

    def _count(i, acc):
        v = buf_v[pl.ds(i * _L, _L)]
        return acc + jnp.where(v > cb, one, zero)

    acc = lax.fori_loop(0, _NV, _count, jnp.zeros((_L,), jnp.float32))

    @pl.when(sid == 0)
    def _ties_add():
        def _ties(i, t):
            v = buf_v[pl.ds(i * _L, _L)]
            g = (i * _L) + lane
            eq = jnp.where(v == cb, one, zero)
            lt42 = jnp.where(g < _CLASS, one, zero)
            return t + eq * lt42

        tie = lax.fori_loop(0, 3, _ties, jnp.zeros((_L,), jnp.float32))
        buf_v[pl.ds(_CHUNK, _L)] = acc + tie

    @pl.when(sid != 0)
    def _plain():
        buf_v[pl.ds(_CHUNK, _L)] = acc

    pltpu.sync_copy(buf_v.at[pl.ds(_CHUNK, _L)], out_hbm.at[pl.ds(sid * _L, _L)])


def kernel(score):
    parts = _judge_sc(score)
    rank = jnp.sum(parts)
    c = score[0, _CLASS]
    has_cough = rank < 10.0
    in_range = (c >= jnp.float32(_MIN)) & (c <= jnp.float32(_MAX))
    judgement = in_range | has_cough
    point = jnp.where(in_range, jnp.float32(_MATCHED_POINT),
                      jnp.where(has_cough, jnp.float32(1.0), jnp.float32(0.0)))
    return judgement, point


# speedup vs baseline: 1.0557x; 1.0557x over previous
import functools

import jax
import jax.numpy as jnp
from jax import lax
from jax.experimental import pallas as pl
from jax.experimental.pallas import tpu as pltpu
from jax.experimental.pallas import tpu_sc as plsc

_N = 8192
_NT = 16
_CHUNK = _N // _NT
_L = 16
_NV = _CHUNK // _L

_CLASS = 42
_MIN = 0.1
_MAX = 1.0
_MATCHED_POINT = 1.5


@functools.partial(
    pl.kernel,
    out_type=jax.ShapeDtypeStruct((_L,), jnp.float32),
    mesh=plsc.VectorSubcoreMesh(
        core_axis_name="c", subcore_axis_name="s", num_cores=1
    ),
    scratch_types=[
        pltpu.VMEM((_CHUNK + _L,), jnp.float32),
        pltpu.VMEM_SHARED((_NT * _L,), jnp.float32),
        pltpu.SemaphoreType.DMA,
    ],
)
def _judge_sc(score_hbm, out_hbm, buf_v, shared, sem):
    sid = lax.axis_index("s")
    base = sid * _CHUNK

    cp = pltpu.async_copy(
        score_hbm.at[0, pl.ds(base, _CHUNK)], buf_v.at[pl.ds(0, _CHUNK)], sem)
    pltpu.sync_copy(score_hbm.at[0, pl.ds(40, _L)],
                    buf_v.at[pl.ds(_CHUNK, _L)])

    lane = lax.iota(jnp.int32, _L)
    cv = buf_v[pl.ds(_CHUNK, _L)]
    cb = cv.at[jnp.full((_L,), _CLASS - 40, jnp.int32)].get(
        mode="promise_in_bounds")

    cp.wait()

    one = jnp.full((_L,), 1.0, jnp.float32)
    zero = jnp.zeros((_L,), jnp.float32)

    def _count(i, acc):
        v = buf_v[pl.ds(i * _L, _L)]
        return acc + jnp.where(v > cb, one, zero)

    acc = lax.fori_loop(0, _NV, _count, jnp.zeros((_L,), jnp.float32))
    buf_v[pl.ds(_CHUNK, _L)] = acc

    pltpu.sync_copy(buf_v.at[pl.ds(_CHUNK, _L)], shared.at[pl.ds(sid * _L, _L)])
    plsc.subcore_barrier()

    @pl.when(sid == 0)
    def _finish():
        def _ties(i, acc):
            v = buf_v[pl.ds(i * _L, _L)]
            g = (i * _L) + lane
            eq = jnp.where(v == cb, one, zero)
            lt42 = jnp.where(g < _CLASS, one, zero)
            return acc + eq * lt42

        tie = lax.fori_loop(0, 3, _ties, jnp.zeros((_L,), jnp.float32))

        pltpu.sync_copy(shared, buf_v.at[pl.ds(0, _NT * _L)])

        def _merge(t, tot):
            return tot + buf_v[pl.ds(t * _L, _L)]

        tot = lax.fori_loop(0, _NT, _merge, tie)
        for sh in (8, 4, 2, 1):
            tot = tot + tot.at[(lane + sh) & (_L - 1)].get(
                mode="promise_in_bounds")
        rank = tot

        hc = jnp.where(rank < 10.0, one, zero)
        inr = (jnp.where(cb >= _MIN, one, zero)
               * jnp.where(cb <= _MAX, one, zero))
        point = inr * _MATCHED_POINT + (one - inr) * hc
        buf_v[pl.ds(_CHUNK, _L)] = jnp.where(lane == 0, point, zero)
        pltpu.sync_copy(buf_v.at[pl.ds(_CHUNK, _L)], out_hbm)


def kernel(score):
    out = _judge_sc(score)
    point = out[0]
    return point > 0.5, point


# speedup vs baseline: 1.0707x; 1.0142x over previous
import functools

import jax
import jax.numpy as jnp
from jax import lax
from jax.experimental import pallas as pl
from jax.experimental.pallas import tpu as pltpu
from jax.experimental.pallas import tpu_sc as plsc

_N = 8192
_NT = 16
_CHUNK = _N // _NT
_L = 16
_NV = _CHUNK // _L

_CLASS = 42
_MIN = 0.1
_MAX = 1.0
_MATCHED_POINT = 1.5


@functools.partial(
    pl.kernel,
    out_type=jax.ShapeDtypeStruct((_L,), jnp.float32),
    mesh=plsc.VectorSubcoreMesh(
        core_axis_name="c", subcore_axis_name="s", num_cores=1
    ),
    scratch_types=[
        pltpu.VMEM((_CHUNK + _L,), jnp.float32),
        pltpu.VMEM_SHARED(((_NT - 1) * _L,), jnp.float32),
        pltpu.SemaphoreType.DMA,
    ],
)
def _judge_sc(score_hbm, out_hbm, buf_v, shared, sem):
    sid = lax.axis_index("s")
    base = sid * _CHUNK

    cp = pltpu.async_copy(
        score_hbm.at[0, pl.ds(base, _CHUNK)], buf_v.at[pl.ds(0, _CHUNK)], sem)
    pltpu.sync_copy(score_hbm.at[0, pl.ds(40, _L)],
                    buf_v.at[pl.ds(_CHUNK, _L)])

    lane = lax.iota(jnp.int32, _L)
    cv = buf_v[pl.ds(_CHUNK, _L)]
    cb = cv.at[jnp.full((_L,), _CLASS - 40, jnp.int32)].get(
        mode="promise_in_bounds")

    cp.wait()

    one = jnp.full((_L,), 1.0, jnp.float32)
    zero = jnp.zeros((_L,), jnp.float32)

    def _count(i, acc):
        v0 = buf_v[pl.ds(i * 2 * _L, _L)]
        v1 = buf_v[pl.ds((i * 2 + 1) * _L, _L)]
        return (acc + jnp.where(v0 > cb, one, zero)
                + jnp.where(v1 > cb, one, zero))

    acc = lax.fori_loop(0, _NV // 2, _count, jnp.zeros((_L,), jnp.float32))

    @pl.when(sid != 0)
    def _stage():
        buf_v[pl.ds(_CHUNK, _L)] = acc
        pltpu.sync_copy(buf_v.at[pl.ds(_CHUNK, _L)],
                        shared.at[pl.ds((sid - 1) * _L, _L)])

    def _ties(i, t):
        v = buf_v[pl.ds(i * _L, _L)]
        g = (i * _L) + lane
        eq = jnp.where(v == cb, one, zero)
        lt42 = jnp.where(g < _CLASS, one, zero)
        return t + eq * lt42

    tie = lax.fori_loop(0, 3, _ties, acc)

    plsc.subcore_barrier()

    @pl.when(sid == 0)
    def _finish():
        pltpu.sync_copy(shared, buf_v.at[pl.ds(0, (_NT - 1) * _L)])

        def _merge(t, tot):
            return tot + buf_v[pl.ds(t * _L, _L)]

        tot = lax.fori_loop(0, _NT - 1, _merge, tie)
        for sh in (8, 4, 2, 1):
            tot = tot + tot.at[(lane + sh) & (_L - 1)].get(
                mode="promise_in_bounds")
        rank = tot

        hc = jnp.where(rank < 10.0, one, zero)
        inr = (jnp.where(cb >= _MIN, one, zero)
               * jnp.where(cb <= _MAX, one, zero))
        point = inr * _MATCHED_POINT + (one - inr) * hc
        buf_v[pl.ds(_CHUNK, _L)] = jnp.where(lane == 0, point, zero)
        pltpu.sync_copy(buf_v.at[pl.ds(_CHUNK, _L)], out_hbm)


def kernel(score):
    out = _judge_sc(score)
    point = out[0]
    return point > 0.5, point
